# baseline (speedup 1.0000x reference)
import functools

import jax
import jax.numpy as jnp
from jax import lax
from jax.experimental import pallas as pl
from jax.experimental.pallas import tpu as pltpu
from jax.experimental.pallas import tpu_sc as plsc

EMBED = 768
ROWS = 4 * 8192
NC = 2
NS = 16
ROWS_PER_SC = ROWS // NC
ROWS_PER_W = ROWS_PER_SC // NS
REP = 64
VREGS_PER_ROW = EMBED // 16
TILE_WORDS = REP * EMBED
STAGE_ROWS = REP * NS
STAGE_WORDS = STAGE_ROWS * EMBED
W_WORDS = ROWS_PER_W * EMBED


def _sc_fill(tok_hbm, out_hbm, tok_v, buf_v, stage_s, sem):
    cid = lax.axis_index("c")
    sid = lax.axis_index("s")
    pltpu.sync_copy(tok_hbm, tok_v)

    row = [tok_v[pl.ds(j * 16, 16)] for j in range(VREGS_PER_ROW)]

    def rep_body(r, carry):
        for j in range(VREGS_PER_ROW):
            buf_v[pl.ds(r * EMBED + j * 16, 16)] = row[j]
        return carry

    lax.fori_loop(0, REP, rep_body, 0)

    pltpu.sync_copy(buf_v, stage_s.at[pl.ds(sid * TILE_WORDS, TILE_WORDS)])
    plsc.subcore_barrier()

    base = (cid * NS + sid) * W_WORDS
    pltpu.async_copy(stage_s, out_hbm.at[pl.ds(base, W_WORDS)], sem).wait()


def kernel(x, mask_token):
    del x
    tok = mask_token.reshape(EMBED)
    mesh = plsc.VectorSubcoreMesh(core_axis_name="c", subcore_axis_name="s")
    fill = functools.partial(
        pl.kernel,
        mesh=mesh,
        out_type=jax.ShapeDtypeStruct((ROWS * EMBED,), jnp.float32),
        scratch_types=[
            pltpu.VMEM((EMBED,), jnp.float32),
            pltpu.VMEM((TILE_WORDS,), jnp.float32),
            pltpu.VMEM_SHARED((STAGE_WORDS,), jnp.float32),
            pltpu.SemaphoreType.DMA,
        ],
    )(_sc_fill)
    out = fill(tok)
    return out.reshape(4, 8192, EMBED)


# speedup vs baseline: 5.3985x; 5.3985x over previous
import jax
import jax.numpy as jnp
from jax.experimental import pallas as pl
from jax.experimental.pallas import tpu as pltpu

EMBED = 768
TOTAL_ROWS = 4 * 8192
BLOCK_ROWS = 1024
N_COPIES = TOTAL_ROWS // BLOCK_ROWS


def _fill_body(tok_ref, out_hbm, scratch, sem):
    scratch[...] = jnp.broadcast_to(tok_ref[...], scratch.shape)
    copies = [
        pltpu.make_async_copy(
            scratch, out_hbm.at[pl.ds(i * BLOCK_ROWS, BLOCK_ROWS), :], sem
        )
        for i in range(N_COPIES)
    ]
    for c in copies:
        c.start()
    for c in copies:
        c.wait()


def kernel(x, mask_token):
    del x
    out = pl.pallas_call(
        _fill_body,
        in_specs=[pl.BlockSpec(memory_space=pltpu.VMEM)],
        out_specs=pl.BlockSpec(memory_space=pl.ANY),
        out_shape=jax.ShapeDtypeStruct((TOTAL_ROWS, EMBED), mask_token.dtype),
        scratch_shapes=[
            pltpu.VMEM((BLOCK_ROWS, EMBED), jnp.float32),
            pltpu.SemaphoreType.DMA,
        ],
    )(mask_token)
    return out.reshape(4, 8192, EMBED)
